# baseline (speedup 1.0000x reference)
import jax
import jax.numpy as jnp
from jax import lax
from jax.experimental import pallas as pl
from jax.experimental.pallas import tpu as pltpu
from jax.experimental.pallas import tpu_sc as plsc

_MASKED_TOKEN = 0
_NUM_CORES = 2
_NUM_SUBCORES = 16
_NUM_WORKERS = _NUM_CORES * _NUM_SUBCORES
_BB = 128
_LANES = 16
_NG = _BB // _LANES
_DEPTH = 4


def _gather_body(idx_hbm, table_hbm, out_hbm,
                 idxblk_v, cols_v, cols2_v, rows_v, tp0_v, tp1_v, semg, semw):
    t_len, d, nb = out_hbm.shape
    blocks = nb // _BB
    blocks_per_w = blocks // _NUM_WORKERS
    wid = lax.axis_index("s") * _NUM_CORES + lax.axis_index("c")

    lane = lax.iota(jnp.int32, _LANES)
    tok_base = [(g * _LANES + lane) * t_len for g in range(_NG)]
    toks = [g * _LANES + lane for g in range(_NG)]
    zero16 = jnp.zeros((_LANES,), jnp.int32)
    tps = [tp0_v, tp1_v]

    def fill_col(t, col_v, col2_v):
        for g in range(_NG):
            vec = plsc.load_gather(idxblk_v, [tok_base[g] + t])
            col_v[pl.ds(g * _LANES, _LANES)] = vec
            col2_v[pl.ds(g * _LANES, _LANES)] = vec >> 1

    def transpose(col_v, row_v, tp_v):
        masks = []
        offs = []
        for g in range(_NG):
            vec = col_v[pl.ds(g * _LANES, _LANES)]
            masks.append(jnp.where(vec == _MASKED_TOKEN, 0.0, 1.0))
            offs.append((vec & 1) * d)

        @plsc.parallel_loop(0, d // 4, unroll=4)
        def _(q):
            d0 = q * 4
            dvec0 = zero16 + d0
            for k in range(4):
                dvec = dvec0 + k
                for g in range(_NG):
                    tp_v[d0 + k, pl.ds(g * _LANES, _LANES)] = (
                        plsc.load_gather(row_v, [toks[g], dvec + offs[g]])
                        * masks[g])

    def block_step(bi, carry):
        blk = wid * blocks_per_w + bi
        pltpu.sync_copy(idx_hbm.at[pl.ds(blk * _BB * t_len, _BB * t_len)],
                        idxblk_v)

        for r in range(_DEPTH):
            fill_col(r, cols_v.at[r], cols2_v.at[r])
            pltpu.async_copy(table_hbm.at[cols2_v.at[r]], rows_v.at[r], semg)

        def token_quad(i, carry2):
            for r in range(_DEPTH):
                t = _DEPTH * i + r
                p = r % 2
                pltpu.make_async_copy(
                    table_hbm.at[cols2_v.at[r]], rows_v.at[r], semg).wait()

                @pl.when(_DEPTH * i + r >= 2)
                def _(p=p, t=t, blk=blk):
                    pltpu.make_async_copy(
                        tps[p],
                        out_hbm.at[jnp.maximum(t - 2, 0), :,
                                   pl.ds(blk * _BB, _BB)],
                        semw).wait()

                transpose(cols_v.at[r], rows_v.at[r], tps[p])
                pltpu.async_copy(
                    tps[p], out_hbm.at[t, :, pl.ds(blk * _BB, _BB)], semw)

                tn = jnp.minimum(t + _DEPTH, t_len - 1)
                fill_col(tn, cols_v.at[r], cols2_v.at[r])
                pltpu.async_copy(
                    table_hbm.at[cols2_v.at[r]], rows_v.at[r], semg)
            return carry2

        lax.fori_loop(0, t_len // _DEPTH, token_quad, 0)

        for r in range(_DEPTH):
            pltpu.make_async_copy(
                table_hbm.at[cols2_v.at[r]], rows_v.at[r], semg).wait()
        for p in range(2):
            pltpu.make_async_copy(
                tps[p], out_hbm.at[0, :, pl.ds(blk * _BB, _BB)], semw).wait()
        return carry

    lax.fori_loop(0, blocks_per_w, block_step, 0)


def kernel(indices, embed):
    nb, t = indices.shape
    v, d = embed.shape
    flat_idx = indices.reshape(nb * t).astype(jnp.int32)
    table = embed.reshape(v // 2, 2 * d)
    mesh = plsc.VectorSubcoreMesh(
        core_axis_name="c",
        subcore_axis_name="s",
        num_cores=_NUM_CORES,
        num_subcores=_NUM_SUBCORES,
    )
    run = pl.kernel(
        _gather_body,
        out_type=jax.ShapeDtypeStruct((t, d, nb), jnp.float32),
        mesh=mesh,
        scratch_types=[
            pltpu.VMEM((_BB * t,), jnp.int32),
            pltpu.VMEM((_DEPTH, _BB), jnp.int32),
            pltpu.VMEM((_DEPTH, _BB), jnp.int32),
            pltpu.VMEM((_DEPTH, _BB, 2 * d), jnp.float32),
            pltpu.VMEM((d, _BB), jnp.float32),
            pltpu.VMEM((d, _BB), jnp.float32),
            pltpu.SemaphoreType.DMA,
            pltpu.SemaphoreType.DMA,
        ],
        compiler_params=pltpu.CompilerParams(
            needs_layout_passes=False, use_tc_tiling_on_sc=True
        ),
    )
    out = run(flat_idx, table)
    return jnp.transpose(out, (2, 0, 1))
